# Initial kernel scaffold; baseline (speedup 1.0000x reference)
import jax
import jax.numpy as jnp
from jax.experimental import pallas as pl


def kernel(t, emb_table, W, b):
    raise NotImplementedError("write your pallas kernel here")


# speedup vs baseline: 1.6537x; 1.6537x over previous
import jax
import jax.numpy as jnp
from jax import lax
from jax.experimental import pallas as pl
from jax.experimental.pallas import tpu as pltpu
from jax.experimental.pallas import tpu_sc as plsc

N_HID = 256
E = 160000
NC = 2
NS = 16
NW = NC * NS
BPW = E // NW
CH = 128
NF = BPW // CH
TS = BPW - NF * CH


def _fuse_body(emb_ref, w_ref, b_ref, out_ref):
    out_ref[...] = lax.dot_general(
        emb_ref[...], w_ref[...],
        (((1,), (1,)), ((), ())),
        preferred_element_type=jnp.float32,
        precision=lax.Precision.HIGHEST,
    ) + b_ref[...]


def _fuse_table(emb_table, W, b):
    m, n = emb_table.shape
    return pl.pallas_call(
        _fuse_body,
        out_shape=jax.ShapeDtypeStruct((m, n), jnp.float32),
    )(emb_table, W, b.reshape(1, n))


def _gather_body(table_hbm, idx_hbm, out_hbm, idx_v, rows_v, sem0, sem1):
    wid = lax.axis_index("s") * NC + lax.axis_index("c")
    base = pl.multiple_of(wid * BPW, 8)
    pltpu.sync_copy(idx_hbm.at[pl.ds(base, BPW)], idx_v)

    def gather(off, n, buf, sem):
        return pltpu.make_async_copy(
            table_hbm.at[idx_v.at[pl.ds(pl.multiple_of(off, 8), n)]],
            rows_v.at[buf, pl.ds(0, n)], sem)

    def write(off, n, buf):
        pltpu.sync_copy(
            rows_v.at[buf, pl.ds(0, n)],
            out_hbm.at[pl.ds(pl.multiple_of(base + off, 8), n)])

    gather(0, CH, 0, sem0).start()

    def outer(g, carry):
        c0 = 2 * g
        o0 = c0 * CH
        gather(o0 + CH, CH, 1, sem1).start()
        gather(o0, CH, 0, sem0).wait()
        write(o0, CH, 0)
        gather(o0 + 2 * CH, CH, 0, sem0).start()
        gather(o0 + CH, CH, 1, sem1).wait()
        write(o0 + CH, CH, 1)
        return carry

    lax.fori_loop(0, (NF - 1) // 2, outer, 0)

    o_last = (NF - 1) * CH
    gather(NF * CH, TS, 1, sem1).start()
    gather(o_last, CH, 0, sem0).wait()
    write(o_last, CH, 0)
    gather(NF * CH, TS, 1, sem1).wait()
    write(NF * CH, TS, 1)


def _sc_gather(table, t):
    mesh = plsc.VectorSubcoreMesh(
        core_axis_name="c", subcore_axis_name="s",
        num_cores=NC, num_subcores=NS)
    return pl.kernel(
        _gather_body,
        out_type=jax.ShapeDtypeStruct((E, N_HID), jnp.float32),
        mesh=mesh,
        scratch_types=[
            pltpu.VMEM((BPW,), jnp.int32),
            pltpu.VMEM((2, CH, N_HID), jnp.float32),
            pltpu.SemaphoreType.DMA,
            pltpu.SemaphoreType.DMA,
        ],
    )(table, t)


def kernel(t, emb_table, W, b):
    fused = _fuse_table(emb_table, W, b)
    return _sc_gather(fused, t)
